# baseline (speedup 1.0000x reference)
import functools

import jax
import jax.numpy as jnp
from jax import lax
from jax.experimental import pallas as pl
from jax.experimental.pallas import tpu as pltpu
from jax.experimental.pallas import tpu_sc as plsc

_NUM_LABELS = 3
_D = 1024
_B = 16384
_NC = 2
_NS = 16
_NW = _NC * _NS

_SPLIT = 8192
_BSC = _B - _SPLIT
_BPW = _BSC // _NW
_TBLK = 512


def _sc_embedding_lookup(ids, table):
    mesh = plsc.VectorSubcoreMesh(core_axis_name="c", subcore_axis_name="s")

    @functools.partial(
        pl.kernel,
        mesh=mesh,
        out_type=jax.ShapeDtypeStruct((_BSC, _D), jnp.float32),
        scratch_types=[
            pltpu.VMEM((_BPW,), jnp.int32),
            pltpu.VMEM((_NUM_LABELS, _D), jnp.float32),
            pltpu.VMEM((1, _D), jnp.float32),
            pltpu.SemaphoreType.DMA,
        ],
    )
    def k(ids_hbm, table_hbm, out_hbm, idx_v, table_v, dummy_v, sem):
        wid = lax.axis_index("s") * _NC + lax.axis_index("c")
        base = wid * _BPW
        pltpu.sync_copy(ids_hbm.at[pl.ds(_SPLIT + base, _BPW)], idx_v)
        pltpu.sync_copy(table_hbm, table_v)

        def issue_group(g, carry):
            ids16 = idx_v[pl.ds(g * 16, 16)]
            for j in range(16):
                rid = ids16[j]
                pltpu.async_copy(
                    table_v.at[pl.ds(rid, 1)],
                    out_hbm.at[pl.ds(base + g * 16 + j, 1)],
                    sem,
                )
            return carry

        lax.fori_loop(0, _BPW // 16, issue_group, 0)

        def drain(r, carry):
            pltpu.make_async_copy(table_hbm.at[pl.ds(0, 1)], dummy_v, sem).wait()
            return carry

        lax.fori_loop(0, _BPW, drain, 0)

    return k(ids, table)


def _tc_body(ids_ref, table_ref, out_ref):
    ids = ids_ref[0, 0, :].reshape(_TBLK, 1)
    t = table_ref[...]
    r0 = t[0, :].reshape(1, _D)
    r1 = t[1, :].reshape(1, _D)
    r2 = t[2, :].reshape(1, _D)
    out_ref[...] = jnp.where(ids == 0, r0, jnp.where(ids == 1, r1, r2))


def _tc_embedding_lookup(ids, table):
    nblk = _SPLIT // _TBLK
    ids3 = ids[:_SPLIT].reshape(nblk, 1, _TBLK)
    return pl.pallas_call(
        _tc_body,
        grid=(nblk,),
        in_specs=[
            pl.BlockSpec((1, 1, _TBLK), lambda i: (i, 0, 0)),
            pl.BlockSpec((_NUM_LABELS, _D), lambda i: (0, 0)),
        ],
        out_specs=pl.BlockSpec((_TBLK, _D), lambda i: (i, 0)),
        out_shape=jax.ShapeDtypeStruct((_SPLIT, _D), jnp.float32),
        compiler_params=pltpu.CompilerParams(
            dimension_semantics=("arbitrary",),
        ),
    )(ids3, table)


def kernel(sentiment_ids, embedding_table):
    ids = sentiment_ids.astype(jnp.int32)
    table = embedding_table.astype(jnp.float32)
    top = _tc_embedding_lookup(ids, table)
    bot = _sc_embedding_lookup(ids, table)
    return jnp.concatenate([top, bot], axis=0)


# speedup vs baseline: 1.9217x; 1.9217x over previous
import functools

import jax
import jax.numpy as jnp
from jax import lax
from jax.experimental import pallas as pl
from jax.experimental.pallas import tpu as pltpu
from jax.experimental.pallas import tpu_sc as plsc

_NUM_LABELS = 3
_D = 1024
_B = 16384
_NC = 2
_NS = 16
_NW = _NC * _NS
_BPW = _B // _NW
_DRAIN = 64


def _sc_embedding_lookup(ids, table):
    mesh = plsc.VectorSubcoreMesh(core_axis_name="c", subcore_axis_name="s")

    @functools.partial(
        pl.kernel,
        mesh=mesh,
        out_type=jax.ShapeDtypeStruct((_B, _D), jnp.float32),
        scratch_types=[
            pltpu.VMEM((_BPW,), jnp.int32),
            pltpu.VMEM((_NUM_LABELS, _D), jnp.float32),
            pltpu.VMEM((_DRAIN, _D), jnp.float32),
            pltpu.SemaphoreType.DMA,
        ],
    )
    def k(ids_hbm, table_hbm, out_hbm, idx_v, table_v, dummy_v, sem):
        wid = lax.axis_index("s") * _NC + lax.axis_index("c")
        base = wid * _BPW
        pltpu.sync_copy(ids_hbm.at[pl.ds(base, _BPW)], idx_v)
        pltpu.sync_copy(table_hbm, table_v)

        def issue_group(g, carry):
            ids16 = idx_v[pl.ds(g * 16, 16)]
            for j in range(16):
                rid = ids16[j]
                pltpu.async_copy(
                    table_v.at[pl.ds(rid, 1)],
                    out_hbm.at[pl.ds(base + g * 16 + j, 1)],
                    sem,
                )
            return carry

        lax.fori_loop(0, _BPW // 16, issue_group, 0)

        def drain(r, carry):
            pltpu.make_async_copy(out_hbm.at[pl.ds(0, _DRAIN)], dummy_v, sem).wait()
            return carry

        lax.fori_loop(0, _BPW // _DRAIN, drain, 0)

    return k(ids, table)


def kernel(sentiment_ids, embedding_table):
    ids = sentiment_ids.astype(jnp.int32)
    return _sc_embedding_lookup(ids, embedding_table.astype(jnp.float32))


# speedup vs baseline: 2.0043x; 1.0430x over previous
import functools

import jax
import jax.numpy as jnp
from jax import lax
from jax.experimental import pallas as pl
from jax.experimental.pallas import tpu as pltpu
from jax.experimental.pallas import tpu_sc as plsc

_NUM_LABELS = 3
_D = 1024
_B = 16384
_NC = 2
_NS = 16
_NW = _NC * _NS
_BPW0 = 480
_BPW1 = 544
_DRAIN = 32


def _sc_embedding_lookup(ids, table):
    mesh = plsc.VectorSubcoreMesh(core_axis_name="c", subcore_axis_name="s")

    @functools.partial(
        pl.kernel,
        mesh=mesh,
        out_type=jax.ShapeDtypeStruct((_B, _D), jnp.float32),
        scratch_types=[
            pltpu.VMEM((_BPW1,), jnp.int32),
            pltpu.VMEM((_NUM_LABELS, _D), jnp.float32),
            pltpu.VMEM((_DRAIN, _D), jnp.float32),
            pltpu.SemaphoreType.DMA,
        ],
    )
    def k(ids_hbm, table_hbm, out_hbm, idx_v, table_v, dummy_v, sem):
        c = lax.axis_index("c")
        s = lax.axis_index("s")
        bpw = jnp.where(c == 0, _BPW0, _BPW1)
        base = jnp.where(c == 0, s * _BPW0, _NS * _BPW0 + s * _BPW1)
        pltpu.sync_copy(ids_hbm.at[pl.ds(base, _BPW1)], idx_v)
        pltpu.sync_copy(table_hbm, table_v)

        def issue_group(g, carry):
            ids16 = idx_v[pl.ds(g * 16, 16)]
            for j in range(16):
                rid = ids16[j]
                pltpu.async_copy(
                    table_v.at[pl.ds(rid, 1)],
                    out_hbm.at[pl.ds(base + g * 16 + j, 1)],
                    sem,
                )
            return carry

        lax.fori_loop(0, bpw // 16, issue_group, 0)

        def drain(r, carry):
            pltpu.make_async_copy(out_hbm.at[pl.ds(0, _DRAIN)], dummy_v, sem).wait()
            return carry

        lax.fori_loop(0, bpw // _DRAIN, drain, 0)

    return k(ids, table)


def kernel(sentiment_ids, embedding_table):
    ids = sentiment_ids.astype(jnp.int32)
    return _sc_embedding_lookup(ids, embedding_table.astype(jnp.float32))


# speedup vs baseline: 2.0558x; 1.0257x over previous
import functools

import jax
import jax.numpy as jnp
from jax import lax
from jax.experimental import pallas as pl
from jax.experimental.pallas import tpu as pltpu
from jax.experimental.pallas import tpu_sc as plsc

_NUM_LABELS = 3
_D = 1024
_B = 16384
_NC = 2
_NS = 16
_NW = _NC * _NS
_BPW0 = 448
_BPW1 = 576
_DRAIN = 32


def _sc_embedding_lookup(ids, table):
    mesh = plsc.VectorSubcoreMesh(core_axis_name="c", subcore_axis_name="s")

    @functools.partial(
        pl.kernel,
        mesh=mesh,
        out_type=jax.ShapeDtypeStruct((_B, _D), jnp.float32),
        scratch_types=[
            pltpu.VMEM((_BPW1,), jnp.int32),
            pltpu.VMEM((_NUM_LABELS, _D), jnp.float32),
            pltpu.VMEM((_DRAIN, _D), jnp.float32),
            pltpu.SemaphoreType.DMA,
        ],
    )
    def k(ids_hbm, table_hbm, out_hbm, idx_v, table_v, dummy_v, sem):
        c = lax.axis_index("c")
        s = lax.axis_index("s")
        bpw = jnp.where(c == 0, _BPW0, _BPW1)
        base = jnp.where(c == 0, s * _BPW0, _NS * _BPW0 + s * _BPW1)
        pltpu.sync_copy(ids_hbm.at[pl.ds(base, _BPW1)], idx_v)
        pltpu.sync_copy(table_hbm, table_v)

        def issue_group(g, carry):
            ids16 = idx_v[pl.ds(g * 16, 16)]
            for j in range(16):
                rid = ids16[j]
                pltpu.async_copy(
                    table_v.at[pl.ds(rid, 1)],
                    out_hbm.at[pl.ds(base + g * 16 + j, 1)],
                    sem,
                )
            return carry

        lax.fori_loop(0, bpw // 16, issue_group, 0)

        def drain(r, carry):
            pltpu.make_async_copy(out_hbm.at[pl.ds(0, _DRAIN)], dummy_v, sem).wait()
            return carry

        lax.fori_loop(0, bpw // _DRAIN, drain, 0)

    return k(ids, table)


def kernel(sentiment_ids, embedding_table):
    ids = sentiment_ids.astype(jnp.int32)
    return _sc_embedding_lookup(ids, embedding_table.astype(jnp.float32))


# speedup vs baseline: 2.0656x; 1.0048x over previous
import functools

import jax
import jax.numpy as jnp
from jax import lax
from jax.experimental import pallas as pl
from jax.experimental.pallas import tpu as pltpu
from jax.experimental.pallas import tpu_sc as plsc

_NUM_LABELS = 3
_D = 1024
_B = 16384
_NC = 2
_NS = 16
_NW = _NC * _NS
_BPW0 = 416
_BPW1 = 608
_DRAIN = 32


def _sc_embedding_lookup(ids, table):
    mesh = plsc.VectorSubcoreMesh(core_axis_name="c", subcore_axis_name="s")

    @functools.partial(
        pl.kernel,
        mesh=mesh,
        out_type=jax.ShapeDtypeStruct((_B, _D), jnp.float32),
        scratch_types=[
            pltpu.VMEM((_BPW1,), jnp.int32),
            pltpu.VMEM((_NUM_LABELS, _D), jnp.float32),
            pltpu.VMEM((_DRAIN, _D), jnp.float32),
            pltpu.SemaphoreType.DMA,
        ],
    )
    def k(ids_hbm, table_hbm, out_hbm, idx_v, table_v, dummy_v, sem):
        c = lax.axis_index("c")
        s = lax.axis_index("s")
        bpw = jnp.where(c == 0, _BPW0, _BPW1)
        base = jnp.where(c == 0, s * _BPW0, _NS * _BPW0 + s * _BPW1)
        pltpu.sync_copy(ids_hbm.at[pl.ds(base, _BPW1)], idx_v)
        pltpu.sync_copy(table_hbm, table_v)

        def issue_group(g, carry):
            ids16 = idx_v[pl.ds(g * 16, 16)]
            for j in range(16):
                rid = ids16[j]
                pltpu.async_copy(
                    table_v.at[pl.ds(rid, 1)],
                    out_hbm.at[pl.ds(base + g * 16 + j, 1)],
                    sem,
                )
            return carry

        lax.fori_loop(0, bpw // 16, issue_group, 0)

        def drain(r, carry):
            pltpu.make_async_copy(out_hbm.at[pl.ds(0, _DRAIN)], dummy_v, sem).wait()
            return carry

        lax.fori_loop(0, bpw // _DRAIN, drain, 0)

    return k(ids, table)


def kernel(sentiment_ids, embedding_table):
    ids = sentiment_ids.astype(jnp.int32)
    return _sc_embedding_lookup(ids, embedding_table.astype(jnp.float32))
